# initial kernel scaffold (unmeasured)
import jax
import jax.numpy as jnp
from jax import lax
from jax.experimental import pallas as pl
from jax.experimental.pallas import tpu as pltpu

N_DEV = 16
M_PER = 64
D = 512
H_PER = 1024


def kernel(x, Win0, Wout0, Win1, Wout1, Win2, Wout2):
    def body(x_ref, win0, wout0, win1, wout1, win2, wout2, out_ref,
             xg_ref, part_ref, rs_ref, ag_sems, rs_sems, snd_sems):
        my_i = lax.axis_index("i")

        def all_gather():
            rdmas = []
            for d in range(1, N_DEV):
                tgt = lax.rem(my_i + d, N_DEV)
                rdma = pltpu.make_async_remote_copy(
                    src_ref=xg_ref.at[pl.ds(my_i * M_PER, M_PER), :],
                    dst_ref=xg_ref.at[pl.ds(my_i * M_PER, M_PER), :],
                    send_sem=snd_sems.at[d],
                    recv_sem=ag_sems.at[my_i],
                    device_id=(tgt,),
                    device_id_type=pl.DeviceIdType.MESH,
                )
                rdma.start()
                rdmas.append(rdma)
            for d in range(1, N_DEV):
                src_j = lax.rem(my_i + d, N_DEV)
                recv = pltpu.make_async_remote_copy(
                    src_ref=xg_ref.at[pl.ds(src_j * M_PER, M_PER), :],
                    dst_ref=xg_ref.at[pl.ds(src_j * M_PER, M_PER), :],
                    send_sem=snd_sems.at[d],
                    recv_sem=ag_sems.at[src_j],
                    device_id=(src_j,),
                    device_id_type=pl.DeviceIdType.MESH,
                )
                recv.wait_recv()
            for r in rdmas:
                r.wait_send()

        def reduce_scatter():
            rdmas = []
            for d in range(1, N_DEV):
                tgt = lax.rem(my_i + d, N_DEV)
                rdma = pltpu.make_async_remote_copy(
                    src_ref=part_ref.at[pl.ds(tgt * M_PER, M_PER), :],
                    dst_ref=rs_ref.at[my_i],
                    send_sem=snd_sems.at[d],
                    recv_sem=rs_sems.at[my_i],
                    device_id=(tgt,),
                    device_id_type=pl.DeviceIdType.MESH,
                )
                rdma.start()
                rdmas.append(rdma)
            rs_ref[my_i] = part_ref[pl.ds(my_i * M_PER, M_PER), :]
            for d in range(1, N_DEV):
                src_j = lax.rem(my_i + d, N_DEV)
                recv = pltpu.make_async_remote_copy(
                    src_ref=part_ref.at[pl.ds(src_j * M_PER, M_PER), :],
                    dst_ref=rs_ref.at[src_j],
                    send_sem=snd_sems.at[d],
                    recv_sem=rs_sems.at[src_j],
                    device_id=(src_j,),
                    device_id_type=pl.DeviceIdType.MESH,
                )
                recv.wait_recv()
            for r in rdmas:
                r.wait_send()

        xg_ref[pl.ds(my_i * M_PER, M_PER), :] = x_ref[:, :]
        all_gather()
        for layer, (wi, wo) in enumerate(
            [(win0, wout0), (win1, wout1), (win2, wout2)]
        ):
            h = jnp.maximum(
                jnp.dot(xg_ref[:, :], wi[:, :],
                        preferred_element_type=jnp.float32),
                0.0,
            )
            part_ref[:, :] = jnp.dot(
                h, wo[:, :], preferred_element_type=jnp.float32
            )
            reduce_scatter()
            new_x = jnp.sum(rs_ref[:, :, :], axis=0)
            if layer < 2:
                xg_ref[pl.ds(my_i * M_PER, M_PER), :] = new_x
                all_gather()
            else:
                out_ref[:, :] = new_x

    return pl.pallas_call(
        body,
        out_shape=jax.ShapeDtypeStruct((M_PER, D), jnp.float32),
        in_specs=[pl.BlockSpec(memory_space=pltpu.VMEM)] * 7,
        out_specs=pl.BlockSpec(memory_space=pltpu.VMEM),
        scratch_shapes=[
            pltpu.VMEM((N_DEV * M_PER, D), jnp.float32),
            pltpu.VMEM((N_DEV * M_PER, D), jnp.float32),
            pltpu.VMEM((N_DEV, M_PER, D), jnp.float32),
            pltpu.SemaphoreType.DMA((N_DEV,)),
            pltpu.SemaphoreType.DMA((N_DEV,)),
            pltpu.SemaphoreType.DMA((N_DEV,)),
        ],
        compiler_params=pltpu.CompilerParams(collective_id=0),
    )(x, Win0, Wout0, Win1, Wout1, Win2, Wout2)


# baseline (device time: 171826 ns/iter reference)
import jax
import jax.numpy as jnp
from jax import lax
from jax.experimental import pallas as pl
from jax.experimental.pallas import tpu as pltpu

N_DEV = 16
M_PER = 64
D = 512
H_PER = 1024


def kernel(x, Win0, Wout0, Win1, Wout1, Win2, Wout2):
    def body(x_ref, win0, wout0, win1, wout1, win2, wout2, out_ref,
             xg_ref, part_ref, rs_ref, ag_sems, rs_sems, snd_sems):
        my_i = lax.axis_index("i")

        def all_gather():
            rdmas = []
            for d in range(1, N_DEV):
                tgt = lax.rem(my_i + d, N_DEV)
                rdma = pltpu.make_async_remote_copy(
                    src_ref=xg_ref.at[pl.ds(my_i * M_PER, M_PER), :],
                    dst_ref=xg_ref.at[pl.ds(my_i * M_PER, M_PER), :],
                    send_sem=snd_sems.at[d],
                    recv_sem=ag_sems.at[my_i],
                    device_id=(tgt,),
                    device_id_type=pl.DeviceIdType.MESH,
                )
                rdma.start()
                rdmas.append(rdma)
            for d in range(1, N_DEV):
                src_j = lax.rem(my_i + d, N_DEV)
                recv = pltpu.make_async_remote_copy(
                    src_ref=xg_ref.at[pl.ds(src_j * M_PER, M_PER), :],
                    dst_ref=xg_ref.at[pl.ds(src_j * M_PER, M_PER), :],
                    send_sem=snd_sems.at[d],
                    recv_sem=ag_sems.at[src_j],
                    device_id=(src_j,),
                    device_id_type=pl.DeviceIdType.MESH,
                )
                recv.wait_recv()
            for r in rdmas:
                r.wait_send()

        def reduce_scatter():
            rdmas = []
            for d in range(1, N_DEV):
                tgt = lax.rem(my_i + d, N_DEV)
                rdma = pltpu.make_async_remote_copy(
                    src_ref=part_ref.at[pl.ds(tgt * M_PER, M_PER), :],
                    dst_ref=rs_ref.at[my_i],
                    send_sem=snd_sems.at[d],
                    recv_sem=rs_sems.at[my_i],
                    device_id=(tgt,),
                    device_id_type=pl.DeviceIdType.MESH,
                )
                rdma.start()
                rdmas.append(rdma)
            rs_ref[my_i] = part_ref[pl.ds(my_i * M_PER, M_PER), :]
            for d in range(1, N_DEV):
                src_j = lax.rem(my_i + d, N_DEV)
                recv = pltpu.make_async_remote_copy(
                    src_ref=part_ref.at[pl.ds(src_j * M_PER, M_PER), :],
                    dst_ref=rs_ref.at[src_j],
                    send_sem=snd_sems.at[d],
                    recv_sem=rs_sems.at[src_j],
                    device_id=(src_j,),
                    device_id_type=pl.DeviceIdType.MESH,
                )
                recv.wait_recv()
            for r in rdmas:
                r.wait_send()

        xg_ref[pl.ds(my_i * M_PER, M_PER), :] = x_ref[:, :]
        all_gather()
        for layer, (wi, wo) in enumerate(
            [(win0, wout0), (win1, wout1), (win2, wout2)]
        ):
            h = jnp.maximum(
                jnp.dot(xg_ref[:, :], wi[:, :],
                        preferred_element_type=jnp.float32),
                0.0,
            )
            part_ref[:, :] = jnp.dot(
                h, wo[:, :], preferred_element_type=jnp.float32
            )
            reduce_scatter()
            new_x = jnp.sum(rs_ref[:, :, :], axis=0)
            if layer < 2:
                xg_ref[pl.ds(my_i * M_PER, M_PER), :] = new_x
                all_gather()
            else:
                out_ref[:, :] = new_x

    return pl.pallas_call(
        body,
        out_shape=jax.ShapeDtypeStruct((M_PER, D), jnp.float32),
        in_specs=[pl.BlockSpec(memory_space=pltpu.VMEM)] * 7,
        out_specs=pl.BlockSpec(memory_space=pltpu.VMEM),
        scratch_shapes=[
            pltpu.VMEM((N_DEV * M_PER, D), jnp.float32),
            pltpu.VMEM((N_DEV * M_PER, D), jnp.float32),
            pltpu.VMEM((N_DEV, M_PER, D), jnp.float32),
            pltpu.SemaphoreType.DMA((N_DEV,)),
            pltpu.SemaphoreType.DMA((N_DEV,)),
            pltpu.SemaphoreType.DMA((N_DEV,)),
        ],
    )(x, Win0, Wout0, Win1, Wout1, Win2, Wout2)


# device time: 157226 ns/iter; 1.0929x vs baseline; 1.0929x over previous
import jax
import jax.numpy as jnp
from jax import lax
from jax.experimental import pallas as pl
from jax.experimental.pallas import tpu as pltpu

N_DEV = 16
M_PER = 64
D = 512
H_PER = 1024
G = 4
NG = N_DEV // G


def kernel(x, Win0, Wout0, Win1, Wout1, Win2, Wout2):
    def body(x_ref, win0, wout0, win1, wout1, win2, wout2, out_ref,
             xg_ref, part_ref, rs_ref, ag_sems, rs_sems, ag_snd, rs_snd):
        my_i = lax.axis_index("i")

        xg_ref[pl.ds(0, M_PER), :] = x_ref[:, :]

        for layer, (wi, wo) in enumerate(
            [(win0, wout0), (win1, wout1), (win2, wout2)]
        ):
            ag_rdmas = []
            for d in range(1, N_DEV):
                tgt = lax.rem(my_i + d, N_DEV)
                rdma = pltpu.make_async_remote_copy(
                    src_ref=xg_ref.at[pl.ds(0, M_PER), :],
                    dst_ref=xg_ref.at[pl.ds((N_DEV - d) * M_PER, M_PER), :],
                    send_sem=ag_snd.at[d],
                    recv_sem=ag_sems.at[my_i],
                    device_id=(tgt,),
                    device_id_type=pl.DeviceIdType.MESH,
                )
                rdma.start()
                ag_rdmas.append(rdma)

            rs_rdmas = []
            for g in range(NG):
                for d in range(g * G, (g + 1) * G):
                    if d == 0:
                        continue
                    src_j = lax.rem(my_i + d, N_DEV)
                    recv = pltpu.make_async_remote_copy(
                        src_ref=xg_ref.at[pl.ds(d * M_PER, M_PER), :],
                        dst_ref=xg_ref.at[pl.ds(d * M_PER, M_PER), :],
                        send_sem=ag_snd.at[d],
                        recv_sem=ag_sems.at[src_j],
                        device_id=(src_j,),
                        device_id_type=pl.DeviceIdType.MESH,
                    )
                    recv.wait_recv()
                rows = pl.ds(g * G * M_PER, G * M_PER)
                h = jnp.maximum(
                    jnp.dot(xg_ref[rows, :], wi[:, :],
                            preferred_element_type=jnp.float32),
                    0.0,
                )
                part_ref[rows, :] = jnp.dot(
                    h, wo[:, :], preferred_element_type=jnp.float32
                )
                for d in range(g * G, (g + 1) * G):
                    if d == 0:
                        continue
                    tgt = lax.rem(my_i + d, N_DEV)
                    rdma = pltpu.make_async_remote_copy(
                        src_ref=part_ref.at[pl.ds(d * M_PER, M_PER), :],
                        dst_ref=rs_ref.at[my_i],
                        send_sem=rs_snd.at[d],
                        recv_sem=rs_sems.at[my_i],
                        device_id=(tgt,),
                        device_id_type=pl.DeviceIdType.MESH,
                    )
                    rdma.start()
                    rs_rdmas.append(rdma)

            rs_ref[my_i] = part_ref[pl.ds(0, M_PER), :]

            for d in range(1, N_DEV):
                src_j = lax.rem(my_i + d, N_DEV)
                recv = pltpu.make_async_remote_copy(
                    src_ref=part_ref.at[pl.ds(d * M_PER, M_PER), :],
                    dst_ref=rs_ref.at[src_j],
                    send_sem=rs_snd.at[d],
                    recv_sem=rs_sems.at[src_j],
                    device_id=(src_j,),
                    device_id_type=pl.DeviceIdType.MESH,
                )
                recv.wait_recv()
            new_x = jnp.sum(rs_ref[:, :, :], axis=0)
            for r in ag_rdmas:
                r.wait_send()
            for r in rs_rdmas:
                r.wait_send()
            if layer < 2:
                xg_ref[pl.ds(0, M_PER), :] = new_x
            else:
                out_ref[:, :] = new_x

    return pl.pallas_call(
        body,
        out_shape=jax.ShapeDtypeStruct((M_PER, D), jnp.float32),
        in_specs=[pl.BlockSpec(memory_space=pltpu.VMEM)] * 7,
        out_specs=pl.BlockSpec(memory_space=pltpu.VMEM),
        scratch_shapes=[
            pltpu.VMEM((N_DEV * M_PER, D), jnp.float32),
            pltpu.VMEM((N_DEV * M_PER, D), jnp.float32),
            pltpu.VMEM((N_DEV, M_PER, D), jnp.float32),
            pltpu.SemaphoreType.DMA((N_DEV,)),
            pltpu.SemaphoreType.DMA((N_DEV,)),
            pltpu.SemaphoreType.DMA((N_DEV,)),
            pltpu.SemaphoreType.DMA((N_DEV,)),
        ],
    )(x, Win0, Wout0, Win1, Wout1, Win2, Wout2)


# device time: 157155 ns/iter; 1.0934x vs baseline; 1.0005x over previous
import jax
import jax.numpy as jnp
from jax import lax
from jax.experimental import pallas as pl
from jax.experimental.pallas import tpu as pltpu

N_DEV = 16
M_PER = 64
D = 512
H_PER = 1024
GROUP_SIZES = [1, 4, 4, 4, 2, 1]


def kernel(x, Win0, Wout0, Win1, Wout1, Win2, Wout2):
    def body(x_ref, win0, wout0, win1, wout1, win2, wout2, out_ref,
             xg_ref, part_ref, rs_ref, ag_sems, rs_sems, ag_snd, rs_snd):
        my_i = lax.axis_index("i")

        xg_ref[pl.ds(0, M_PER), :] = x_ref[:, :]

        for layer, (wi, wo) in enumerate(
            [(win0, wout0), (win1, wout1), (win2, wout2)]
        ):
            ag_rdmas = []
            for d in range(1, N_DEV):
                tgt = lax.rem(my_i + d, N_DEV)
                rdma = pltpu.make_async_remote_copy(
                    src_ref=xg_ref.at[pl.ds(0, M_PER), :],
                    dst_ref=xg_ref.at[pl.ds((N_DEV - d) * M_PER, M_PER), :],
                    send_sem=ag_snd.at[d],
                    recv_sem=ag_sems.at[my_i],
                    device_id=(tgt,),
                    device_id_type=pl.DeviceIdType.MESH,
                )
                rdma.start()
                ag_rdmas.append(rdma)

            rs_rdmas = []
            d0 = 0
            for gsz in GROUP_SIZES:
                for d in range(d0, d0 + gsz):
                    if d == 0:
                        continue
                    src_j = lax.rem(my_i + d, N_DEV)
                    recv = pltpu.make_async_remote_copy(
                        src_ref=xg_ref.at[pl.ds(d * M_PER, M_PER), :],
                        dst_ref=xg_ref.at[pl.ds(d * M_PER, M_PER), :],
                        send_sem=ag_snd.at[d],
                        recv_sem=ag_sems.at[src_j],
                        device_id=(src_j,),
                        device_id_type=pl.DeviceIdType.MESH,
                    )
                    recv.wait_recv()
                rows = pl.ds(d0 * M_PER, gsz * M_PER)
                h = jnp.maximum(
                    jnp.dot(xg_ref[rows, :], wi[:, :],
                            preferred_element_type=jnp.float32),
                    0.0,
                )
                part_ref[rows, :] = jnp.dot(
                    h, wo[:, :], preferred_element_type=jnp.float32
                )
                for d in range(d0, d0 + gsz):
                    if d == 0:
                        rs_ref[my_i] = part_ref[pl.ds(0, M_PER), :]
                        continue
                    tgt = lax.rem(my_i + d, N_DEV)
                    rdma = pltpu.make_async_remote_copy(
                        src_ref=part_ref.at[pl.ds(d * M_PER, M_PER), :],
                        dst_ref=rs_ref.at[my_i],
                        send_sem=rs_snd.at[d],
                        recv_sem=rs_sems.at[my_i],
                        device_id=(tgt,),
                        device_id_type=pl.DeviceIdType.MESH,
                    )
                    rdma.start()
                    rs_rdmas.append(rdma)
                d0 += gsz

            for d in range(1, N_DEV):
                src_j = lax.rem(my_i + d, N_DEV)
                recv = pltpu.make_async_remote_copy(
                    src_ref=part_ref.at[pl.ds(d * M_PER, M_PER), :],
                    dst_ref=rs_ref.at[src_j],
                    send_sem=rs_snd.at[d],
                    recv_sem=rs_sems.at[src_j],
                    device_id=(src_j,),
                    device_id_type=pl.DeviceIdType.MESH,
                )
                recv.wait_recv()
            new_x = jnp.sum(rs_ref[:, :, :], axis=0)
            for r in ag_rdmas:
                r.wait_send()
            for r in rs_rdmas:
                r.wait_send()
            if layer < 2:
                xg_ref[pl.ds(0, M_PER), :] = new_x
            else:
                out_ref[:, :] = new_x

    return pl.pallas_call(
        body,
        out_shape=jax.ShapeDtypeStruct((M_PER, D), jnp.float32),
        in_specs=[pl.BlockSpec(memory_space=pltpu.VMEM)] * 7,
        out_specs=pl.BlockSpec(memory_space=pltpu.VMEM),
        scratch_shapes=[
            pltpu.VMEM((N_DEV * M_PER, D), jnp.float32),
            pltpu.VMEM((N_DEV * M_PER, D), jnp.float32),
            pltpu.VMEM((N_DEV, M_PER, D), jnp.float32),
            pltpu.SemaphoreType.DMA((N_DEV,)),
            pltpu.SemaphoreType.DMA((N_DEV,)),
            pltpu.SemaphoreType.DMA((N_DEV,)),
            pltpu.SemaphoreType.DMA((N_DEV,)),
        ],
    )(x, Win0, Wout0, Win1, Wout1, Win2, Wout2)


# device time: 93048 ns/iter; 1.8466x vs baseline; 1.6890x over previous
import jax
import jax.numpy as jnp
from jax import lax
from jax.experimental import pallas as pl
from jax.experimental.pallas import tpu as pltpu

N_DEV = 16
M_PER = 64
D = 512
H_PER = 1024
GROUP_SIZES = [1, 4, 4, 4, 2, 1]


def kernel(x, Win0, Wout0, Win1, Wout1, Win2, Wout2):
    def body(x_ref, win0, wout0, win1, wout1, win2, wout2, out_ref,
             xg_ref, part_ref, rs_ref, ag_sems, rs_sems, ag_snd, rs_snd):
        my_i = lax.axis_index("i")

        xg_ref[pl.ds(0, M_PER), :] = x_ref[:, :].astype(jnp.bfloat16)

        for layer, (wi, wo) in enumerate(
            [(win0, wout0), (win1, wout1), (win2, wout2)]
        ):
            ag_rdmas = []
            for d in range(1, N_DEV):
                tgt = lax.rem(my_i + d, N_DEV)
                rdma = pltpu.make_async_remote_copy(
                    src_ref=xg_ref.at[pl.ds(0, M_PER), :],
                    dst_ref=xg_ref.at[pl.ds((N_DEV - d) * M_PER, M_PER), :],
                    send_sem=ag_snd.at[d],
                    recv_sem=ag_sems.at[my_i],
                    device_id=(tgt,),
                    device_id_type=pl.DeviceIdType.MESH,
                )
                rdma.start()
                ag_rdmas.append(rdma)

            rs_rdmas = []
            d0 = 0
            for gsz in GROUP_SIZES:
                for d in range(d0, d0 + gsz):
                    if d == 0:
                        continue
                    src_j = lax.rem(my_i + d, N_DEV)
                    recv = pltpu.make_async_remote_copy(
                        src_ref=xg_ref.at[pl.ds(d * M_PER, M_PER), :],
                        dst_ref=xg_ref.at[pl.ds(d * M_PER, M_PER), :],
                        send_sem=ag_snd.at[d],
                        recv_sem=ag_sems.at[src_j],
                        device_id=(src_j,),
                        device_id_type=pl.DeviceIdType.MESH,
                    )
                    recv.wait_recv()
                rows = pl.ds(d0 * M_PER, gsz * M_PER)
                h = jnp.maximum(
                    jnp.dot(xg_ref[rows, :].astype(jnp.float32), wi[:, :],
                            preferred_element_type=jnp.float32),
                    0.0,
                )
                part_ref[rows, :] = jnp.dot(
                    h, wo[:, :], preferred_element_type=jnp.float32
                ).astype(jnp.bfloat16)
                for d in range(d0, d0 + gsz):
                    if d == 0:
                        rs_ref[my_i] = part_ref[pl.ds(0, M_PER), :]
                        continue
                    tgt = lax.rem(my_i + d, N_DEV)
                    rdma = pltpu.make_async_remote_copy(
                        src_ref=part_ref.at[pl.ds(d * M_PER, M_PER), :],
                        dst_ref=rs_ref.at[my_i],
                        send_sem=rs_snd.at[d],
                        recv_sem=rs_sems.at[my_i],
                        device_id=(tgt,),
                        device_id_type=pl.DeviceIdType.MESH,
                    )
                    rdma.start()
                    rs_rdmas.append(rdma)
                d0 += gsz

            for d in range(1, N_DEV):
                src_j = lax.rem(my_i + d, N_DEV)
                recv = pltpu.make_async_remote_copy(
                    src_ref=part_ref.at[pl.ds(d * M_PER, M_PER), :],
                    dst_ref=rs_ref.at[src_j],
                    send_sem=rs_snd.at[d],
                    recv_sem=rs_sems.at[src_j],
                    device_id=(src_j,),
                    device_id_type=pl.DeviceIdType.MESH,
                )
                recv.wait_recv()
            new_x = jnp.sum(rs_ref[:, :, :].astype(jnp.float32), axis=0)
            for r in ag_rdmas:
                r.wait_send()
            for r in rs_rdmas:
                r.wait_send()
            if layer < 2:
                xg_ref[pl.ds(0, M_PER), :] = new_x.astype(jnp.bfloat16)
            else:
                out_ref[:, :] = new_x

    return pl.pallas_call(
        body,
        out_shape=jax.ShapeDtypeStruct((M_PER, D), jnp.float32),
        in_specs=[pl.BlockSpec(memory_space=pltpu.VMEM)] * 7,
        out_specs=pl.BlockSpec(memory_space=pltpu.VMEM),
        scratch_shapes=[
            pltpu.VMEM((N_DEV * M_PER, D), jnp.bfloat16),
            pltpu.VMEM((N_DEV * M_PER, D), jnp.bfloat16),
            pltpu.VMEM((N_DEV, M_PER, D), jnp.bfloat16),
            pltpu.SemaphoreType.DMA((N_DEV,)),
            pltpu.SemaphoreType.DMA((N_DEV,)),
            pltpu.SemaphoreType.DMA((N_DEV,)),
            pltpu.SemaphoreType.DMA((N_DEV,)),
        ],
    )(x, Win0, Wout0, Win1, Wout1, Win2, Wout2)
